# baseline (speedup 1.0000x reference)
import functools

import jax
import jax.numpy as jnp
from jax import lax
from jax.experimental import pallas as pl
from jax.experimental.pallas import tpu as pltpu
from jax.experimental.pallas import tpu_sc as plsc

_NC = 2
_NS = 16
_NW = _NC * _NS
_CHUNK = 128


@functools.partial(jax.jit, static_argnames=("rows_per_w",))
def _sc_gather(table, idx_flat, *, rows_per_w):
    n_idx = idx_flat.shape[0]
    d = table.shape[1]
    n_chunks = rows_per_w // _CHUNK
    mesh = plsc.VectorSubcoreMesh(core_axis_name="c", subcore_axis_name="s")

    @functools.partial(
        pl.kernel,
        mesh=mesh,
        out_type=jax.ShapeDtypeStruct((n_idx, d), jnp.float32),
        scratch_types=[
            pltpu.VMEM((rows_per_w,), jnp.int32),
            pltpu.VMEM((_CHUNK, d), jnp.float32),
            pltpu.SemaphoreType.DMA,
        ],
    )
    def k(table_hbm, idx_hbm, out_hbm, idx_v, rows_v, sem):
        wid = lax.axis_index("s") * _NC + lax.axis_index("c")
        base = wid * rows_per_w
        pltpu.sync_copy(idx_hbm.at[pl.ds(base, rows_per_w)], idx_v)

        @pl.loop(0, n_chunks)
        def _(i):
            pltpu.async_copy(
                table_hbm.at[idx_v.at[pl.ds(i * _CHUNK, _CHUNK)]], rows_v, sem
            ).wait()
            pltpu.sync_copy(rows_v, out_hbm.at[pl.ds(base + i * _CHUNK, _CHUNK)])

    return k(table, idx_flat)


_BB = 8


@functools.partial(jax.jit, static_argnames=("s",))
def _tc_unpad(xp, *, s):
    n_rows, sp, d = xp.shape

    def body(in_ref, out_ref):
        out_ref[...] = in_ref[:, :s, :]

    return pl.pallas_call(
        body,
        grid=(n_rows // _BB,),
        in_specs=[pl.BlockSpec((_BB, sp, d), lambda i: (i, 0, 0))],
        out_specs=pl.BlockSpec((_BB, s, d), lambda i: (i, 0, 0)),
        out_shape=jax.ShapeDtypeStruct((n_rows, s, d), jnp.float32),
    )(xp)


def kernel(inputs, kernel):
    table = kernel
    ids = inputs.astype(jnp.int32)
    n_rows, s = ids.shape
    d = table.shape[1]
    sp = -(-s // 8) * 8
    v = table.shape[0]
    pad_ids = (
        jnp.arange(n_rows, dtype=jnp.int32)[:, None] * (sp - s)
        + jnp.arange(sp - s, dtype=jnp.int32)[None, :]
    ) % v
    idsp = jnp.concatenate([ids, pad_ids], axis=1)
    n = n_rows * sp
    assert n % (_NW * _CHUNK) == 0
    out = _sc_gather(table, idsp.reshape(-1), rows_per_w=n // _NW)
    assert n_rows % _BB == 0
    return _tc_unpad(out.reshape(n_rows, sp, d), s=s)


# speedup vs baseline: 1.6409x; 1.6409x over previous
import functools

import jax
import jax.numpy as jnp
from jax import lax
from jax.experimental import pallas as pl
from jax.experimental.pallas import tpu as pltpu
from jax.experimental.pallas import tpu_sc as plsc

_NC = 2
_NS = 16
_NW = _NC * _NS
_CHUNK = 128


@functools.partial(jax.jit, static_argnames=("rows_per_w",))
def _sc_gather(table, idx_flat, *, rows_per_w):
    n_idx = idx_flat.shape[0]
    d = table.shape[1]
    n_chunks = rows_per_w // _CHUNK
    mesh = plsc.VectorSubcoreMesh(core_axis_name="c", subcore_axis_name="s")

    @functools.partial(
        pl.kernel,
        mesh=mesh,
        out_type=jax.ShapeDtypeStruct((n_idx, d), jnp.float32),
        scratch_types=[
            pltpu.VMEM((rows_per_w,), jnp.int32),
            pltpu.VMEM((_CHUNK, d), jnp.float32),
            pltpu.SemaphoreType.DMA,
        ],
    )
    def k(table_hbm, idx_hbm, out_hbm, idx_v, rows_v, sem):
        wid = lax.axis_index("s") * _NC + lax.axis_index("c")
        base = wid * rows_per_w
        pltpu.sync_copy(idx_hbm.at[pl.ds(base, rows_per_w)], idx_v)

        @pl.loop(0, n_chunks)
        def _(i):
            pltpu.async_copy(
                table_hbm.at[idx_v.at[pl.ds(i * _CHUNK, _CHUNK)]], rows_v, sem
            ).wait()
            pltpu.sync_copy(rows_v, out_hbm.at[pl.ds(base + i * _CHUNK, _CHUNK)])

    return k(table, idx_flat)


def kernel(inputs, kernel):
    table = kernel
    ids = inputs.astype(jnp.int32)
    n_rows, s = ids.shape
    d = table.shape[1]
    sp = -(-s // 8) * 8
    v = table.shape[0]
    pad_ids = (
        jnp.arange(n_rows, dtype=jnp.int32)[:, None] * (sp - s)
        + jnp.arange(sp - s, dtype=jnp.int32)[None, :]
    ) % v
    idsp = jnp.concatenate([ids, pad_ids], axis=1)
    n = n_rows * sp
    assert n % (_NW * _CHUNK) == 0
    out = _sc_gather(table, idsp.reshape(-1), rows_per_w=n // _NW)
    return out.reshape(n_rows, sp, d)[:, :s, :]
